# Initial kernel scaffold; baseline (speedup 1.0000x reference)
import jax
import jax.numpy as jnp
from jax.experimental import pallas as pl


def kernel(x, edge_index, edge_attr, batch, linatoms_w, linatoms_b, mes_w1, mes_b1, mes_w2, mes_b2, root_w, conv_b, bn_g, bn_b, pred_w, pred_b):
    raise NotImplementedError("write your pallas kernel here")


# speedup vs baseline: 1.7426x; 1.7426x over previous
import functools

import jax
import jax.numpy as jnp
from jax import lax
from jax.experimental import pallas as pl
from jax.experimental.pallas import tpu as pltpu
from jax.experimental.pallas import tpu_sc as plsc

NC = 2
NS = 16
NW = NC * NS
CH = 128


def _h0_body(x_ref, w_ref, b_ref, o_ref):
    h = jnp.dot(x_ref[...], w_ref[...], preferred_element_type=jnp.float32,
                precision=lax.Precision.HIGHEST)
    h = h + b_ref[...]
    o_ref[...] = jnp.where(h > 0, h, 0.01 * h)


def _msg_body(hs_ref, ea_ref, w1_ref, b1_ref, wp_ref, o_ref, *, neu, c):
    e1 = jnp.dot(ea_ref[...], w1_ref[...], preferred_element_type=jnp.float32,
                precision=lax.Precision.HIGHEST)
    e1 = jnp.maximum(e1 + b1_ref[...], 0.0)
    u = jnp.dot(hs_ref[...], wp_ref[...], preferred_element_type=jnp.float32,
                precision=lax.Precision.HIGHEST)
    msg = u[:, neu * c:(neu + 1) * c]
    for k in range(neu):
        msg = msg + e1[:, k:k + 1] * u[:, k * c:(k + 1) * c]
    o_ref[...] = msg


def _upd_body(agg_ref, h_ref, rw_ref, cb_ref, g_ref, bb_ref, o_ref, *, n,
              n_pad, act):
    a = agg_ref[:n, :] + agg_ref[n_pad:n_pad + n, :]
    hn = a + jnp.dot(h_ref[...], rw_ref[...],
                     preferred_element_type=jnp.float32,
                precision=lax.Precision.HIGHEST) + cb_ref[...]
    mean = jnp.mean(hn, axis=0, keepdims=True)
    var = jnp.mean((hn - mean) ** 2, axis=0, keepdims=True)
    hn = (hn - mean) * lax.rsqrt(var + 1e-5) * g_ref[...] + bb_ref[...]
    if act:
        hn = jnp.where(hn > 0, hn, 0.01 * hn)
    o_ref[...] = hn


def _pool_body(h_ref, batch_ref, pw_ref, pb_ref, o_ref, *, n_blocks, blk, g):
    acc = jnp.zeros((g, h_ref.shape[1]), jnp.float32)
    gid = lax.broadcasted_iota(jnp.int32, (g, blk), 0)
    for j in range(n_blocks):
        oh = (batch_ref[j:j + 1, :] == gid).astype(jnp.float32)
        acc = acc + jnp.dot(oh, h_ref[pl.ds(j * blk, blk), :],
                            preferred_element_type=jnp.float32,
                precision=lax.Precision.HIGHEST)
    o_ref[...] = jnp.sum(acc * pw_ref[...], axis=1, keepdims=True) + pb_ref[...]


@functools.lru_cache(maxsize=None)
def _make_gather(n, c, e_pad, epw, nch):
    mesh = plsc.VectorSubcoreMesh(core_axis_name="c", subcore_axis_name="s")

    @functools.partial(
        pl.kernel,
        out_type=jax.ShapeDtypeStruct((e_pad, c), jnp.float32),
        mesh=mesh,
        scratch_types=[
            pltpu.VMEM((nch, CH), jnp.int32),
            pltpu.VMEM((CH, c), jnp.float32),
            pltpu.SemaphoreType.DMA,
        ],
        compiler_params=pltpu.CompilerParams(use_tc_tiling_on_sc=False),
    )
    def gather_k(h_hbm, src_hbm, out_hbm, idx_v, rows_v, sem):
        cid = lax.axis_index("c")
        sid = lax.axis_index("s")
        wid = sid * NC + cid
        pltpu.sync_copy(src_hbm.at[pl.ds(wid * nch, nch)], idx_v)

        def body(j, carry):
            pltpu.async_copy(h_hbm.at[idx_v.at[j]], rows_v, sem).wait()
            base = pl.multiple_of(wid * epw + j * CH, CH)
            pltpu.sync_copy(rows_v, out_hbm.at[pl.ds(base, CH)])
            return carry

        lax.fori_loop(0, nch, body, 0)

    return gather_k


@functools.lru_cache(maxsize=None)
def _make_scatter(c, n_pad, e_pad, epw, nch, sub_rows):
    mesh = plsc.VectorSubcoreMesh(core_axis_name="c", subcore_axis_name="s")

    @functools.partial(
        pl.kernel,
        out_type=jax.ShapeDtypeStruct((NC * n_pad, c), jnp.float32),
        mesh=mesh,
        scratch_types=[
            pltpu.VMEM((nch, CH), jnp.int32),
            pltpu.VMEM((CH, c), jnp.float32),
            pltpu.VMEM((sub_rows, c), jnp.float32),
            pltpu.VMEM_SHARED((n_pad, c), jnp.float32),
        ],
        compiler_params=pltpu.CompilerParams(use_tc_tiling_on_sc=False),
    )
    def scatter_k(msg_hbm, dst_hbm, out_hbm, idx_v, buf_v, stage_v, agg_sh):
        cid = lax.axis_index("c")
        sid = lax.axis_index("s")
        wid = sid * NC + cid
        zv = jnp.zeros((16,), jnp.float32)

        def zbody(i, carry):
            stage_v[i, :] = zv
            return carry

        lax.fori_loop(0, sub_rows, zbody, 0)
        pltpu.sync_copy(stage_v, agg_sh.at[pl.ds(sid * sub_rows, sub_rows)])
        pltpu.sync_copy(dst_hbm.at[pl.ds(wid * nch, nch)], idx_v)
        plsc.subcore_barrier()

        def body(j, carry):
            base = pl.multiple_of(wid * epw + j * CH, CH)
            pltpu.sync_copy(msg_hbm.at[pl.ds(base, CH)], buf_v)
            pltpu.sync_copy(buf_v, agg_sh.at[idx_v.at[j]], add=True)
            return carry

        lax.fori_loop(0, nch, body, 0)
        plsc.subcore_barrier()
        pltpu.sync_copy(agg_sh.at[pl.ds(sid * sub_rows, sub_rows)],
                        out_hbm.at[pl.ds(cid * n_pad + sid * sub_rows,
                                         sub_rows)])

    return scatter_k


def kernel(x, edge_index, edge_attr, batch, linatoms_w, linatoms_b, mes_w1,
           mes_b1, mes_w2, mes_b2, root_w, conv_b, bn_g, bn_b, pred_w,
           pred_b):
    n, d = x.shape
    e = edge_index.shape[1]
    de = edge_attr.shape[1]
    c = linatoms_w.shape[1]
    nlay, _, neu = mes_w1.shape
    g = 512

    nch = -(-e // (NW * CH))
    epw = nch * CH
    e_pad = NW * epw
    pad = e_pad - e
    n_pad = -(-(n + 1) // NS) * NS
    sub_rows = n_pad // NS

    src2d = jnp.concatenate(
        [edge_index[0], jnp.zeros((pad,), jnp.int32)]).reshape(-1, CH)
    dst2d = jnp.concatenate(
        [edge_index[1], jnp.full((pad,), n, jnp.int32)]).reshape(-1, CH)
    ea_p = jnp.concatenate(
        [edge_attr, jnp.zeros((pad, de), jnp.float32)], axis=0)

    gather_k = _make_gather(n, c, e_pad, epw, nch)
    scatter_k = _make_scatter(c, n_pad, e_pad, epw, nch, sub_rows)

    h = pl.pallas_call(
        _h0_body,
        out_shape=jax.ShapeDtypeStruct((n, c), jnp.float32),
    )(x, linatoms_w, linatoms_b.reshape(1, c))

    blk_e = 2048
    n_eblk = e_pad // blk_e

    for l in range(nlay):
        h_src = gather_k(h, src2d)
        wp = jnp.concatenate(
            [mes_w2[l].reshape(neu, c, c),
             mes_b2[l].reshape(1, c, c)], axis=0).transpose(1, 0, 2)
        wp = wp.reshape(c, (neu + 1) * c)
        msg = pl.pallas_call(
            functools.partial(_msg_body, neu=neu, c=c),
            grid=(n_eblk,),
            in_specs=[
                pl.BlockSpec((blk_e, c), lambda i: (i, 0)),
                pl.BlockSpec((blk_e, de), lambda i: (i, 0)),
                pl.BlockSpec((de, neu), lambda i: (0, 0)),
                pl.BlockSpec((1, neu), lambda i: (0, 0)),
                pl.BlockSpec((c, (neu + 1) * c), lambda i: (0, 0)),
            ],
            out_specs=pl.BlockSpec((blk_e, c), lambda i: (i, 0)),
            out_shape=jax.ShapeDtypeStruct((e_pad, c), jnp.float32),
        )(h_src, ea_p, mes_w1[l], mes_b1[l].reshape(1, neu), wp)
        agg = scatter_k(msg, dst2d)
        h = pl.pallas_call(
            functools.partial(_upd_body, n=n, n_pad=n_pad,
                              act=(l < nlay - 1)),
            out_shape=jax.ShapeDtypeStruct((n, c), jnp.float32),
        )(agg, h, root_w[l], conv_b[l].reshape(1, c), bn_g[l].reshape(1, c),
          bn_b[l].reshape(1, c))

    blk_n = 1
    for cand in range(1024, 7, -1):
        if n % cand == 0 and cand % 8 == 0:
            blk_n = cand
            break
    nb = n // blk_n
    batch2d = batch.reshape(nb, blk_n)
    out = pl.pallas_call(
        functools.partial(_pool_body, n_blocks=nb, blk=blk_n, g=g),
        out_shape=jax.ShapeDtypeStruct((g, 1), jnp.float32),
    )(h, batch2d, pred_w.reshape(1, c), pred_b.reshape(1, 1))
    return out


# speedup vs baseline: 3.4076x; 1.9554x over previous
import functools

import jax
import jax.numpy as jnp
from jax import lax
from jax.experimental import pallas as pl
from jax.experimental.pallas import tpu as pltpu
from jax.experimental.pallas import tpu_sc as plsc

NC = 2
NS = 16
NW = NC * NS
CH = 128


def _pro_body(ea_ref, w1_ref, b1_ref, x_ref, lw_ref, lb_ref, wp_ref,
              ep0_ref, ep1_ref, ep2_ref, h0_ref, t0_ref, *, neu, nlay, blk):
    u = jnp.dot(ea_ref[...], w1_ref[...], preferred_element_type=jnp.float32,
                precision=lax.Precision.HIGHEST)
    u = jnp.maximum(u + b1_ref[...], 0.0)
    one = jnp.ones((blk, 1), jnp.float32)
    zero = jnp.zeros((blk, 16 - neu - 1), jnp.float32)
    for l, ref in enumerate((ep0_ref, ep1_ref, ep2_ref)[:nlay]):
        ref[...] = jnp.concatenate(
            [u[:, l * neu:(l + 1) * neu], one, zero], axis=1)

    @pl.when(pl.program_id(0) == 0)
    def _():
        h = jnp.dot(x_ref[...], lw_ref[...],
                    preferred_element_type=jnp.float32,
                    precision=lax.Precision.HIGHEST) + lb_ref[...]
        h = jnp.where(h > 0, h, 0.01 * h)
        h0_ref[...] = h
        t0_ref[...] = jnp.dot(h, wp_ref[...],
                              preferred_element_type=jnp.float32,
                              precision=lax.Precision.HIGHEST)


def _upd_body(agg_ref, h_ref, rw_ref, cb_ref, g_ref, bb_ref, wp_ref,
              o_ref, t_ref, *, n, n_pad, act):
    a = agg_ref[:n, :] + agg_ref[n_pad:n_pad + n, :]
    hn = a + jnp.dot(h_ref[...], rw_ref[...],
                     preferred_element_type=jnp.float32,
                     precision=lax.Precision.HIGHEST) + cb_ref[...]
    mean = jnp.mean(hn, axis=0, keepdims=True)
    var = jnp.mean((hn - mean) ** 2, axis=0, keepdims=True)
    hn = (hn - mean) * lax.rsqrt(var + 1e-5) * g_ref[...] + bb_ref[...]
    if act:
        hn = jnp.where(hn > 0, hn, 0.01 * hn)
    o_ref[...] = hn
    t_ref[...] = jnp.dot(hn, wp_ref[...], preferred_element_type=jnp.float32,
                         precision=lax.Precision.HIGHEST)


def _last_body(agg_ref, h_ref, rw_ref, cb_ref, g_ref, bb_ref, batch_ref,
               pw_ref, pb_ref, o_ref, *, n, n_pad, n_blocks, blk, ng):
    a = agg_ref[:n, :] + agg_ref[n_pad:n_pad + n, :]
    hn = a + jnp.dot(h_ref[...], rw_ref[...],
                     preferred_element_type=jnp.float32,
                     precision=lax.Precision.HIGHEST) + cb_ref[...]
    mean = jnp.mean(hn, axis=0, keepdims=True)
    var = jnp.mean((hn - mean) ** 2, axis=0, keepdims=True)
    hn = (hn - mean) * lax.rsqrt(var + 1e-5) * g_ref[...] + bb_ref[...]
    acc = jnp.zeros((ng, hn.shape[1]), jnp.float32)
    gid = lax.broadcasted_iota(jnp.int32, (ng, blk), 0)
    for j in range(n_blocks):
        oh = (batch_ref[j:j + 1, :] == gid).astype(jnp.float32)
        acc = acc + jnp.dot(oh, hn[j * blk:(j + 1) * blk, :],
                            preferred_element_type=jnp.float32,
                            precision=lax.Precision.HIGHEST)
    o_ref[...] = jnp.sum(acc * pw_ref[...], axis=1, keepdims=True) + pb_ref[...]


@functools.lru_cache(maxsize=None)
def _make_edge(c, tw, n_pad, e_pad, epw, nch, sub_rows):
    mesh = plsc.VectorSubcoreMesh(core_axis_name="c", subcore_axis_name="s")
    nk = tw // c

    @functools.partial(
        pl.kernel,
        out_type=jax.ShapeDtypeStruct((NC * n_pad, c), jnp.float32),
        mesh=mesh,
        scratch_types=[
            pltpu.VMEM((nch, CH), jnp.int32),
            pltpu.VMEM((nch, CH), jnp.int32),
            pltpu.VMEM((CH, tw), jnp.float32),
            pltpu.VMEM((CH, tw), jnp.float32),
            pltpu.VMEM((CH, 16), jnp.float32),
            pltpu.VMEM((CH, 16), jnp.float32),
            pltpu.VMEM((CH, c), jnp.float32),
            pltpu.VMEM((sub_rows, c), jnp.float32),
            pltpu.VMEM_SHARED((n_pad, c), jnp.float32),
            pltpu.SemaphoreType.DMA,
            pltpu.SemaphoreType.DMA,
        ],
        compiler_params=pltpu.CompilerParams(use_tc_tiling_on_sc=False),
    )
    def edge_k(t_hbm, ep_hbm, src_hbm, dst_hbm, out_hbm, src_v, dst_v,
               tr0_v, tr1_v, ep0_v, ep1_v, msg_v, stage_v, agg_sh,
               sem0, sem1):
        cid = lax.axis_index("c")
        sid = lax.axis_index("s")
        wid = sid * NC + cid
        zv = jnp.zeros((16,), jnp.float32)

        def zbody(i, carry):
            stage_v[i, :] = zv
            return carry

        lax.fori_loop(0, sub_rows, zbody, 0)
        pltpu.sync_copy(stage_v, agg_sh.at[pl.ds(sid * sub_rows, sub_rows)])
        pltpu.sync_copy(src_hbm.at[pl.ds(wid * nch, nch)], src_v)
        pltpu.sync_copy(dst_hbm.at[pl.ds(wid * nch, nch)], dst_v)
        plsc.subcore_barrier()

        def fire(j, tbuf, ebuf, sem):
            pltpu.async_copy(t_hbm.at[src_v.at[j]], tbuf, sem)
            pltpu.async_copy(
                ep_hbm.at[pl.ds(wid * epw + j * CH, CH)], ebuf, sem)

        def process(j, tbuf, ebuf, sem):
            pltpu.make_async_copy(t_hbm.at[src_v.at[j]], tbuf, sem).wait()
            pltpu.make_async_copy(
                ep_hbm.at[pl.ds(wid * epw + j * CH, CH)], ebuf, sem).wait()

            def ebody(e, carry):
                ev = ebuf[e, :]
                msg = tbuf[e, pl.ds((nk - 1) * c, c)]
                for k in range(nk - 1):
                    msg = msg + ev[k] * tbuf[e, pl.ds(k * c, c)]
                msg_v[e, :] = msg
                return carry

            lax.fori_loop(0, CH, ebody, 0)
            pltpu.sync_copy(msg_v, agg_sh.at[dst_v.at[j]], add=True)

        fire(0, tr0_v, ep0_v, sem0)

        def body(t, carry):
            j = t * 2
            fire(j + 1, tr1_v, ep1_v, sem1)
            process(j, tr0_v, ep0_v, sem0)
            fire(j + 2, tr0_v, ep0_v, sem0)
            process(j + 1, tr1_v, ep1_v, sem1)
            return carry

        lax.fori_loop(0, nch // 2 - 1, body, 0)
        fire(nch - 1, tr1_v, ep1_v, sem1)
        process(nch - 2, tr0_v, ep0_v, sem0)
        process(nch - 1, tr1_v, ep1_v, sem1)

        plsc.subcore_barrier()
        pltpu.sync_copy(agg_sh.at[pl.ds(sid * sub_rows, sub_rows)],
                        out_hbm.at[pl.ds(cid * n_pad + sid * sub_rows,
                                         sub_rows)])

    return edge_k


def kernel(x, edge_index, edge_attr, batch, linatoms_w, linatoms_b, mes_w1,
           mes_b1, mes_w2, mes_b2, root_w, conv_b, bn_g, bn_b, pred_w,
           pred_b):
    n, d = x.shape
    e = edge_index.shape[1]
    de = edge_attr.shape[1]
    c = linatoms_w.shape[1]
    nlay, _, neu = mes_w1.shape
    ng = 512
    nk = neu + 1
    tw = nk * c

    nch = -(-e // (NW * CH))
    nch = nch + (nch % 2)
    epw = nch * CH
    e_pad = NW * epw
    pad = e_pad - e
    n_pad = -(-(n + 1) // NS) * NS
    sub_rows = n_pad // NS

    src2d = jnp.concatenate(
        [edge_index[0], jnp.zeros((pad,), jnp.int32)]).reshape(-1, CH)
    dst2d = jnp.concatenate(
        [edge_index[1], jnp.full((pad,), n, jnp.int32)]).reshape(-1, CH)
    ea_p = jnp.concatenate(
        [edge_attr, jnp.zeros((pad, de), jnp.float32)], axis=0)

    def wp_of(l):
        a = jnp.concatenate([mes_w2[l].reshape(neu, c, c),
                             mes_b2[l].reshape(1, c, c)], axis=0)
        return a.transpose(1, 0, 2).reshape(c, tw)

    w1cat = mes_w1.transpose(1, 0, 2).reshape(de, nlay * neu)
    b1cat = mes_b1.reshape(1, nlay * neu)

    edge_k = _make_edge(c, tw, n_pad, e_pad, epw, nch, sub_rows)

    blk_e = 2048
    n_eblk = e_pad // blk_e
    eblk = lambda w: pl.BlockSpec((blk_e, w), lambda i: (i, 0))
    full = lambda s: pl.BlockSpec(s, lambda i: tuple(0 for _ in s))
    ep0, ep1, ep2, h, t = pl.pallas_call(
        functools.partial(_pro_body, neu=neu, nlay=nlay, blk=blk_e),
        grid=(n_eblk,),
        in_specs=[
            eblk(de), full((de, nlay * neu)), full((1, nlay * neu)),
            full((n, d)), full((d, c)), full((1, c)), full((c, tw)),
        ],
        out_specs=[eblk(16), eblk(16), eblk(16), full((n, c)), full((n, tw))],
        out_shape=[
            jax.ShapeDtypeStruct((e_pad, 16), jnp.float32),
            jax.ShapeDtypeStruct((e_pad, 16), jnp.float32),
            jax.ShapeDtypeStruct((e_pad, 16), jnp.float32),
            jax.ShapeDtypeStruct((n, c), jnp.float32),
            jax.ShapeDtypeStruct((n, tw), jnp.float32),
        ],
    )(ea_p, w1cat, b1cat, x, linatoms_w, linatoms_b.reshape(1, c), wp_of(0))
    eps = (ep0, ep1, ep2)

    blk_n = n
    for cand in range(1024, 7, -1):
        if n % cand == 0 and cand % 8 == 0:
            blk_n = cand
            break
    nb = n // blk_n
    batch2d = batch.reshape(nb, blk_n)

    out = None
    for l in range(nlay):
        agg = edge_k(t, eps[l], src2d, dst2d)
        if l < nlay - 1:
            h, t = pl.pallas_call(
                functools.partial(_upd_body, n=n, n_pad=n_pad, act=True),
                out_shape=[jax.ShapeDtypeStruct((n, c), jnp.float32),
                           jax.ShapeDtypeStruct((n, tw), jnp.float32)],
            )(agg, h, root_w[l], conv_b[l].reshape(1, c),
              bn_g[l].reshape(1, c), bn_b[l].reshape(1, c), wp_of(l + 1))
        else:
            out = pl.pallas_call(
                functools.partial(_last_body, n=n, n_pad=n_pad, n_blocks=nb,
                                  blk=blk_n, ng=ng),
                out_shape=jax.ShapeDtypeStruct((ng, 1), jnp.float32),
            )(agg, h, root_w[l], conv_b[l].reshape(1, c),
              bn_g[l].reshape(1, c), bn_b[l].reshape(1, c), batch2d,
              pred_w.reshape(1, c), pred_b.reshape(1, 1))
    return out


# speedup vs baseline: 3.4498x; 1.0124x over previous
import functools

import jax
import jax.numpy as jnp
from jax import lax
from jax.experimental import pallas as pl
from jax.experimental.pallas import tpu as pltpu
from jax.experimental.pallas import tpu_sc as plsc

NC = 2
NS = 16
NW = NC * NS
CH = 128


def _pro_body(ea_ref, w1_ref, b1_ref, x_ref, lw_ref, lb_ref, wp_ref,
              ep0_ref, ep1_ref, ep2_ref, h0_ref, t0_ref, *, neu, nlay, blk):
    u = jnp.dot(ea_ref[...], w1_ref[...], preferred_element_type=jnp.float32,
                precision=lax.Precision.HIGHEST)
    u = jnp.maximum(u + b1_ref[...], 0.0)
    one = jnp.ones((blk, 1), jnp.float32)
    zero = jnp.zeros((blk, 16 - neu - 1), jnp.float32)
    for l, ref in enumerate((ep0_ref, ep1_ref, ep2_ref)[:nlay]):
        ref[...] = jnp.concatenate(
            [u[:, l * neu:(l + 1) * neu], one, zero], axis=1)

    @pl.when(pl.program_id(0) == 0)
    def _():
        h = jnp.dot(x_ref[...], lw_ref[...],
                    preferred_element_type=jnp.float32,
                    precision=lax.Precision.HIGHEST) + lb_ref[...]
        h = jnp.where(h > 0, h, 0.01 * h)
        h0_ref[...] = h
        t0_ref[...] = jnp.dot(h, wp_ref[...],
                              preferred_element_type=jnp.float32,
                              precision=lax.Precision.HIGHEST)


def _upd_body(agg_ref, h_ref, rw_ref, cb_ref, g_ref, bb_ref, wp_ref,
              o_ref, t_ref, *, n, n_pad, act):
    a = agg_ref[:n, :] + agg_ref[n_pad:n_pad + n, :]
    hn = a + jnp.dot(h_ref[...], rw_ref[...],
                     preferred_element_type=jnp.float32,
                     precision=lax.Precision.HIGHEST) + cb_ref[...]
    mean = jnp.mean(hn, axis=0, keepdims=True)
    var = jnp.mean((hn - mean) ** 2, axis=0, keepdims=True)
    hn = (hn - mean) * lax.rsqrt(var + 1e-5) * g_ref[...] + bb_ref[...]
    if act:
        hn = jnp.where(hn > 0, hn, 0.01 * hn)
    o_ref[...] = hn
    t_ref[...] = jnp.dot(hn, wp_ref[...], preferred_element_type=jnp.float32,
                         precision=lax.Precision.HIGHEST)


def _last_body(agg_ref, h_ref, rw_ref, cb_ref, g_ref, bb_ref, batch_ref,
               pw_ref, pb_ref, o_ref, *, n, n_pad, n_blocks, blk, ng):
    a = agg_ref[:n, :] + agg_ref[n_pad:n_pad + n, :]
    hn = a + jnp.dot(h_ref[...], rw_ref[...],
                     preferred_element_type=jnp.float32,
                     precision=lax.Precision.HIGHEST) + cb_ref[...]
    mean = jnp.mean(hn, axis=0, keepdims=True)
    var = jnp.mean((hn - mean) ** 2, axis=0, keepdims=True)
    hn = (hn - mean) * lax.rsqrt(var + 1e-5) * g_ref[...] + bb_ref[...]
    acc = jnp.zeros((ng, hn.shape[1]), jnp.float32)
    gid = lax.broadcasted_iota(jnp.int32, (ng, blk), 0)
    for j in range(n_blocks):
        oh = (batch_ref[j:j + 1, :] == gid).astype(jnp.float32)
        acc = acc + jnp.dot(oh, hn[j * blk:(j + 1) * blk, :],
                            preferred_element_type=jnp.float32,
                            precision=lax.Precision.HIGHEST)
    o_ref[...] = jnp.sum(acc * pw_ref[...], axis=1, keepdims=True) + pb_ref[...]


@functools.lru_cache(maxsize=None)
def _make_edge(c, tw, n_pad, e_pad, epw, nch, sub_rows):
    mesh = plsc.VectorSubcoreMesh(core_axis_name="c", subcore_axis_name="s")
    nk = tw // c

    @functools.partial(
        pl.kernel,
        out_type=jax.ShapeDtypeStruct((NC * n_pad, c), jnp.float32),
        mesh=mesh,
        scratch_types=[
            pltpu.VMEM((nch, CH), jnp.int32),
            pltpu.VMEM((nch, CH), jnp.int32),
            pltpu.VMEM((CH, tw), jnp.float32),
            pltpu.VMEM((CH, tw), jnp.float32),
            pltpu.VMEM((CH, 16), jnp.float32),
            pltpu.VMEM((CH, 16), jnp.float32),
            pltpu.VMEM((CH, c), jnp.float32),
            pltpu.VMEM((sub_rows, c), jnp.float32),
            pltpu.VMEM_SHARED((n_pad, c), jnp.float32),
            pltpu.SemaphoreType.DMA,
            pltpu.SemaphoreType.DMA,
        ],
        compiler_params=pltpu.CompilerParams(use_tc_tiling_on_sc=False),
    )
    def edge_k(t_hbm, ep_hbm, src_hbm, dst_hbm, out_hbm, src_v, dst_v,
               tr0_v, tr1_v, ep0_v, ep1_v, msg_v, stage_v, agg_sh,
               sem0, sem1):
        cid = lax.axis_index("c")
        sid = lax.axis_index("s")
        wid = sid * NC + cid
        zv = jnp.zeros((16,), jnp.float32)

        def zbody(i, carry):
            stage_v[i, :] = zv
            return carry

        lax.fori_loop(0, sub_rows, zbody, 0)
        pltpu.sync_copy(stage_v, agg_sh.at[pl.ds(sid * sub_rows, sub_rows)])
        pltpu.sync_copy(src_hbm.at[pl.ds(wid * nch, nch)], src_v)
        pltpu.sync_copy(dst_hbm.at[pl.ds(wid * nch, nch)], dst_v)
        plsc.subcore_barrier()

        def fire(j, tbuf, ebuf, sem):
            pltpu.async_copy(t_hbm.at[src_v.at[j]], tbuf, sem)
            pltpu.async_copy(
                ep_hbm.at[pl.ds(wid * epw + j * CH, CH)], ebuf, sem)

        def process(j, tbuf, ebuf, sem):
            pltpu.make_async_copy(t_hbm.at[src_v.at[j]], tbuf, sem).wait()
            pltpu.make_async_copy(
                ep_hbm.at[pl.ds(wid * epw + j * CH, CH)], ebuf, sem).wait()

            @plsc.parallel_loop(0, CH, 1, unroll=4)
            def ebody(e):
                ev = ebuf[e, :]
                terms = [ev[k] * tbuf[e, pl.ds(k * c, c)]
                         for k in range(nk - 1)]
                terms.append(tbuf[e, pl.ds((nk - 1) * c, c)])
                while len(terms) > 1:
                    terms = [a + b for a, b in
                             zip(terms[::2], terms[1::2])] + (
                                 [terms[-1]] if len(terms) % 2 else [])
                msg_v[e, :] = terms[0]
            pltpu.sync_copy(msg_v, agg_sh.at[dst_v.at[j]], add=True)

        fire(0, tr0_v, ep0_v, sem0)

        def body(t, carry):
            j = t * 2
            fire(j + 1, tr1_v, ep1_v, sem1)
            process(j, tr0_v, ep0_v, sem0)
            fire(j + 2, tr0_v, ep0_v, sem0)
            process(j + 1, tr1_v, ep1_v, sem1)
            return carry

        lax.fori_loop(0, nch // 2 - 1, body, 0)
        fire(nch - 1, tr1_v, ep1_v, sem1)
        process(nch - 2, tr0_v, ep0_v, sem0)
        process(nch - 1, tr1_v, ep1_v, sem1)

        plsc.subcore_barrier()
        pltpu.sync_copy(agg_sh.at[pl.ds(sid * sub_rows, sub_rows)],
                        out_hbm.at[pl.ds(cid * n_pad + sid * sub_rows,
                                         sub_rows)])

    return edge_k


def kernel(x, edge_index, edge_attr, batch, linatoms_w, linatoms_b, mes_w1,
           mes_b1, mes_w2, mes_b2, root_w, conv_b, bn_g, bn_b, pred_w,
           pred_b):
    n, d = x.shape
    e = edge_index.shape[1]
    de = edge_attr.shape[1]
    c = linatoms_w.shape[1]
    nlay, _, neu = mes_w1.shape
    ng = 512
    nk = neu + 1
    tw = nk * c

    nch = -(-e // (NW * CH))
    nch = nch + (nch % 2)
    epw = nch * CH
    e_pad = NW * epw
    pad = e_pad - e
    n_pad = -(-(n + 1) // NS) * NS
    sub_rows = n_pad // NS

    src2d = jnp.concatenate(
        [edge_index[0], jnp.zeros((pad,), jnp.int32)]).reshape(-1, CH)
    dst2d = jnp.concatenate(
        [edge_index[1], jnp.full((pad,), n, jnp.int32)]).reshape(-1, CH)
    ea_p = jnp.concatenate(
        [edge_attr, jnp.zeros((pad, de), jnp.float32)], axis=0)

    def wp_of(l):
        a = jnp.concatenate([mes_w2[l].reshape(neu, c, c),
                             mes_b2[l].reshape(1, c, c)], axis=0)
        return a.transpose(1, 0, 2).reshape(c, tw)

    w1cat = mes_w1.transpose(1, 0, 2).reshape(de, nlay * neu)
    b1cat = mes_b1.reshape(1, nlay * neu)

    edge_k = _make_edge(c, tw, n_pad, e_pad, epw, nch, sub_rows)

    blk_e = 2048
    n_eblk = e_pad // blk_e
    eblk = lambda w: pl.BlockSpec((blk_e, w), lambda i: (i, 0))
    full = lambda s: pl.BlockSpec(s, lambda i: tuple(0 for _ in s))
    ep0, ep1, ep2, h, t = pl.pallas_call(
        functools.partial(_pro_body, neu=neu, nlay=nlay, blk=blk_e),
        grid=(n_eblk,),
        in_specs=[
            eblk(de), full((de, nlay * neu)), full((1, nlay * neu)),
            full((n, d)), full((d, c)), full((1, c)), full((c, tw)),
        ],
        out_specs=[eblk(16), eblk(16), eblk(16), full((n, c)), full((n, tw))],
        out_shape=[
            jax.ShapeDtypeStruct((e_pad, 16), jnp.float32),
            jax.ShapeDtypeStruct((e_pad, 16), jnp.float32),
            jax.ShapeDtypeStruct((e_pad, 16), jnp.float32),
            jax.ShapeDtypeStruct((n, c), jnp.float32),
            jax.ShapeDtypeStruct((n, tw), jnp.float32),
        ],
    )(ea_p, w1cat, b1cat, x, linatoms_w, linatoms_b.reshape(1, c), wp_of(0))
    eps = (ep0, ep1, ep2)

    blk_n = n
    for cand in range(1024, 7, -1):
        if n % cand == 0 and cand % 8 == 0:
            blk_n = cand
            break
    nb = n // blk_n
    batch2d = batch.reshape(nb, blk_n)

    out = None
    for l in range(nlay):
        agg = edge_k(t, eps[l], src2d, dst2d)
        if l < nlay - 1:
            h, t = pl.pallas_call(
                functools.partial(_upd_body, n=n, n_pad=n_pad, act=True),
                out_shape=[jax.ShapeDtypeStruct((n, c), jnp.float32),
                           jax.ShapeDtypeStruct((n, tw), jnp.float32)],
            )(agg, h, root_w[l], conv_b[l].reshape(1, c),
              bn_g[l].reshape(1, c), bn_b[l].reshape(1, c), wp_of(l + 1))
        else:
            out = pl.pallas_call(
                functools.partial(_last_body, n=n, n_pad=n_pad, n_blocks=nb,
                                  blk=blk_n, ng=ng),
                out_shape=jax.ShapeDtypeStruct((ng, 1), jnp.float32),
            )(agg, h, root_w[l], conv_b[l].reshape(1, c),
              bn_g[l].reshape(1, c), bn_b[l].reshape(1, c), batch2d,
              pred_w.reshape(1, c), pred_b.reshape(1, 1))
    return out
